# baseline (speedup 1.0000x reference)
import functools

import jax
import jax.numpy as jnp
from jax import lax
from jax.experimental import pallas as pl
from jax.experimental.pallas import tpu as pltpu
from jax.experimental.pallas import tpu_sc as plsc

SEQ = 200
SEQ_PAD = 208
CHUNK = 104
NCHUNK = SEQ_PAD // CHUNK

NC, NS = 2, 16
NW = NC * NS

EMBED = 128
LANES = 16
EWORDS = EMBED // 2
WVECS = EWORDS // LANES
EVECS = EMBED // LANES


NBUF = 8


def _pool_body(emb_hbm, idx_hbm, out_hbm, idx_v, rows_v, out_v, *sems):
    bpw = out_v.shape[0]
    wid = lax.axis_index("s") * NC + lax.axis_index("c")
    base = pl.multiple_of(wid * (bpw * SEQ_PAD), 8)
    pltpu.sync_copy(idx_hbm.at[pl.ds(base, bpw * SEQ_PAD)], idx_v)
    nchunks = bpw * NCHUNK

    def start(c, buf):
        off = pl.multiple_of(c * CHUNK, 8)
        pltpu.make_async_copy(
            emb_hbm.at[idx_v.at[pl.ds(off, CHUNK)]],
            rows_v.at[buf], sems[buf]).start()

    def wait(c, buf):
        off = pl.multiple_of(c * CHUNK, 8)
        pltpu.make_async_copy(
            emb_hbm.at[idx_v.at[pl.ds(off, CHUNK)]],
            rows_v.at[buf], sems[buf]).wait()

    for c in range(NBUF):
        start(c, c)

    pairs_per_iter = NBUF // NCHUNK

    def pair_body(p, carry):
        for bb in range(pairs_per_iter):
            b = p * pairs_per_iter + bb
            acc = tuple(jnp.zeros((LANES,), jnp.float32)
                        for _ in range(EVECS))
            for j in range(NCHUNK):
                jj = bb * NCHUNK + j
                c = p * NBUF + jj
                wait(c, jj)

                def row_body(r, a):
                    new = []
                    for k in range(WVECS):
                        w = rows_v[jj, r, pl.ds(k * LANES, LANES)]
                        lo = lax.bitcast_convert_type(w << 16, jnp.float32)
                        hi = lax.bitcast_convert_type(w, jnp.float32)
                        new.append(a[2 * k] + lo)
                        new.append(a[2 * k + 1] + hi)
                    return tuple(new)

                acc = lax.fori_loop(0, CHUNK, row_body, acc, unroll=4)

                @pl.when(c + NBUF < nchunks)
                def _():
                    start(c + NBUF, jj)

            for k in range(EVECS):
                out_v[b, pl.ds(k * LANES, LANES)] = acc[k]
        return carry

    lax.fori_loop(0, nchunks // NBUF, pair_body, 0)
    pltpu.sync_copy(out_v, out_hbm.at[pl.ds(wid * bpw, bpw)])


def _pool(emb_pad, idx_flat, batch):
    bpw = batch // NW
    mesh = plsc.VectorSubcoreMesh(core_axis_name="c", subcore_axis_name="s")
    return pl.kernel(
        _pool_body,
        mesh=mesh,
        compiler_params=pltpu.CompilerParams(use_tc_tiling_on_sc=False),
        out_type=jax.ShapeDtypeStruct((batch, EMBED), jnp.float32),
        scratch_types=[
            pltpu.VMEM((bpw * SEQ_PAD,), jnp.int32),
            pltpu.VMEM((NBUF, CHUNK, EWORDS), jnp.int32),
            pltpu.VMEM((bpw, EMBED), jnp.float32),
        ] + [pltpu.SemaphoreType.DMA] * NBUF,
    )(emb_pad, idx_flat)


def _mlp_body(m_ref, w1_ref, b1_ref, w2_ref, b2_ref, out_ref, *, bm, on, nt):
    m = m_ref[...].astype(jnp.float32) * (1.0 / SEQ)
    h = (jnp.dot(m, w1_ref[...], preferred_element_type=jnp.float32)
         + b1_ref[...]).astype(jnp.bfloat16)
    mx = jnp.full((bm, 1), -1e30, jnp.float32)
    for t in range(nt):
        sl = pl.ds(t * on, on)
        z = (jnp.dot(h, w2_ref[:, sl], preferred_element_type=jnp.float32)
             + b2_ref[:, sl])
        out_ref[:, sl] = z
        mx = jnp.maximum(mx, jnp.max(z, axis=1, keepdims=True))
    s = jnp.zeros((bm, 1), jnp.float32)
    for t in range(nt):
        sl = pl.ds(t * on, on)
        s = s + jnp.sum(jnp.exp(out_ref[:, sl] - mx), axis=1, keepdims=True)
    off = mx + jnp.log(s)
    for t in range(nt):
        sl = pl.ds(t * on, on)
        out_ref[:, sl] = out_ref[:, sl] - off


def _mlp(m, W1, b1r, W2b, b2p, out_cols):
    batch, embed = m.shape
    hidden = W1.shape[1]
    opad = W2b.shape[1]
    bm = 256
    nb = batch // bm
    on = 1280
    nt = opad // on
    return pl.pallas_call(
        functools.partial(_mlp_body, bm=bm, on=on, nt=nt),
        grid=(nb,),
        in_specs=[
            pl.BlockSpec((bm, embed), lambda b: (b, 0)),
            pl.BlockSpec((embed, hidden), lambda b: (0, 0)),
            pl.BlockSpec((1, hidden), lambda b: (0, 0)),
            pl.BlockSpec((hidden, opad), lambda b: (0, 0)),
            pl.BlockSpec((1, opad), lambda b: (0, 0)),
        ],
        out_specs=pl.BlockSpec((bm, opad), lambda b: (b, 0)),
        out_shape=jax.ShapeDtypeStruct((batch, out_cols), jnp.float32),
        compiler_params=pltpu.CompilerParams(
            dimension_semantics=("parallel",)),
    )(m, W1, b1r, W2b, b2p)


def kernel(x, emb, W1, b1, W2, b2):
    seq, batch = x.shape
    vocab, embed = emb.shape
    out_cols = W2.shape[1]

    npad_rows = 64
    pad_idx = vocab + (
        jnp.arange(batch * (SEQ_PAD - seq), dtype=jnp.int32)
        .reshape(batch, SEQ_PAD - seq) % npad_rows)
    xT = jnp.concatenate([x.astype(jnp.int32).T, pad_idx], axis=1)
    idx_flat = xT.reshape(-1)
    emb_pad = jnp.pad(emb.astype(jnp.bfloat16), ((0, npad_rows), (0, 0)))
    emb_packed = jax.lax.bitcast_convert_type(
        emb_pad.reshape(vocab + npad_rows, EWORDS, 2), jnp.int32)

    lane = jnp.arange(LANES)
    perm = jnp.concatenate(
        [jnp.concatenate([32 * k + 2 * lane, 32 * k + 2 * lane + 1])
         for k in range(WVECS)])
    W1p = W1[perm, :]

    opad = ((out_cols + 1279) // 1280) * 1280
    W2b = jnp.pad(W2, ((0, 0), (0, opad - out_cols))).astype(jnp.bfloat16)
    b2p = jnp.pad(b2, (0, opad - out_cols),
                  constant_values=-1e30).reshape(1, -1)
    b1r = b1.reshape(1, -1)

    nsplit = 2
    bchunk = batch // nsplit
    idx2 = idx_flat.reshape(nsplit, bchunk * SEQ_PAD)
    outs = []
    for i in range(nsplit):
        sums_i = _pool(emb_packed, idx2[i], bchunk)
        outs.append(_mlp(sums_i, W1p, b1r, W2b, b2p, out_cols))
    return jnp.concatenate(outs, axis=0)


# speedup vs baseline: 1.0724x; 1.0724x over previous
import functools

import jax
import jax.numpy as jnp
from jax import lax
from jax.experimental import pallas as pl
from jax.experimental.pallas import tpu as pltpu
from jax.experimental.pallas import tpu_sc as plsc

SEQ = 200
SEQ_PAD = 208
CHUNK = 104
NCHUNK = SEQ_PAD // CHUNK

NC, NS = 2, 16
NW = NC * NS

EMBED = 128
LANES = 16
EWORDS = EMBED // 2
WVECS = EWORDS // LANES
EVECS = EMBED // LANES


NBUF = 8


def _pool_body(emb_hbm, idx_hbm, out_hbm, idx_v, rows_v, out_v, *sems):
    bpw = out_v.shape[0]
    wid = lax.axis_index("s") * NC + lax.axis_index("c")
    base = pl.multiple_of(wid * (bpw * SEQ_PAD), 8)
    pltpu.sync_copy(idx_hbm.at[pl.ds(base, bpw * SEQ_PAD)], idx_v)
    nchunks = bpw * NCHUNK

    def start(c, buf):
        off = pl.multiple_of(c * CHUNK, 8)
        pltpu.make_async_copy(
            emb_hbm.at[idx_v.at[pl.ds(off, CHUNK)]],
            rows_v.at[buf], sems[buf]).start()

    def wait(c, buf):
        off = pl.multiple_of(c * CHUNK, 8)
        pltpu.make_async_copy(
            emb_hbm.at[idx_v.at[pl.ds(off, CHUNK)]],
            rows_v.at[buf], sems[buf]).wait()

    for c in range(NBUF):
        start(c, c)

    pairs_per_iter = NBUF // NCHUNK

    def pair_body(p, carry):
        for bb in range(pairs_per_iter):
            b = p * pairs_per_iter + bb
            acc = tuple(jnp.zeros((LANES,), jnp.float32)
                        for _ in range(EVECS))
            for j in range(NCHUNK):
                jj = bb * NCHUNK + j
                c = p * NBUF + jj
                wait(c, jj)

                def row_body(r, a):
                    new = []
                    for k in range(WVECS):
                        w = rows_v[jj, r, pl.ds(k * LANES, LANES)]
                        lo = lax.bitcast_convert_type(w << 16, jnp.float32)
                        hi = lax.bitcast_convert_type(w, jnp.float32)
                        new.append(a[2 * k] + lo)
                        new.append(a[2 * k + 1] + hi)
                    return tuple(new)

                acc = lax.fori_loop(0, CHUNK, row_body, acc, unroll=4)

                @pl.when(c + NBUF < nchunks)
                def _():
                    start(c + NBUF, jj)

            for k in range(EVECS):
                out_v[b, pl.ds(k * LANES, LANES)] = acc[k]
        return carry

    lax.fori_loop(0, nchunks // NBUF, pair_body, 0)
    pltpu.sync_copy(out_v, out_hbm.at[pl.ds(wid * bpw, bpw)])


def _pool(emb_pad, idx_flat, batch):
    bpw = batch // NW
    mesh = plsc.VectorSubcoreMesh(core_axis_name="c", subcore_axis_name="s")
    return pl.kernel(
        _pool_body,
        mesh=mesh,
        compiler_params=pltpu.CompilerParams(use_tc_tiling_on_sc=False),
        out_type=jax.ShapeDtypeStruct((batch, EMBED), jnp.float32),
        scratch_types=[
            pltpu.VMEM((bpw * SEQ_PAD,), jnp.int32),
            pltpu.VMEM((NBUF, CHUNK, EWORDS), jnp.int32),
            pltpu.VMEM((bpw, EMBED), jnp.float32),
        ] + [pltpu.SemaphoreType.DMA] * NBUF,
    )(emb_pad, idx_flat)


def _mlp_body(m_ref, w1_ref, b1_ref, w2_ref, b2_ref, out_ref, *, bm, on, nt):
    m = m_ref[...].astype(jnp.float32) * (1.0 / SEQ)
    h = (jnp.dot(m, w1_ref[...], preferred_element_type=jnp.float32)
         + b1_ref[...]).astype(jnp.bfloat16)
    mx = jnp.full((bm, 1), -1e30, jnp.float32)
    for t in range(nt):
        sl = pl.ds(t * on, on)
        z = (jnp.dot(h, w2_ref[:, sl], preferred_element_type=jnp.float32)
             + b2_ref[:, sl])
        out_ref[:, sl] = z
        mx = jnp.maximum(mx, jnp.max(z, axis=1, keepdims=True))
    s = jnp.zeros((bm, 1), jnp.float32)
    for t in range(nt):
        sl = pl.ds(t * on, on)
        s = s + jnp.sum(jnp.exp(out_ref[:, sl] - mx), axis=1, keepdims=True)
    off = mx + jnp.log(s)
    for t in range(nt):
        sl = pl.ds(t * on, on)
        out_ref[:, sl] = out_ref[:, sl] - off


def _mlp(m, W1, b1r, W2b, b2p, out_cols):
    batch, embed = m.shape
    hidden = W1.shape[1]
    opad = W2b.shape[1]
    bm = 256
    nb = batch // bm
    on = 1280
    nt = opad // on
    return pl.pallas_call(
        functools.partial(_mlp_body, bm=bm, on=on, nt=nt),
        grid=(nb,),
        in_specs=[
            pl.BlockSpec((bm, embed), lambda b: (b, 0)),
            pl.BlockSpec((embed, hidden), lambda b: (0, 0)),
            pl.BlockSpec((1, hidden), lambda b: (0, 0)),
            pl.BlockSpec((hidden, opad), lambda b: (0, 0)),
            pl.BlockSpec((1, opad), lambda b: (0, 0)),
        ],
        out_specs=pl.BlockSpec((bm, opad), lambda b: (b, 0)),
        out_shape=jax.ShapeDtypeStruct((batch, out_cols), jnp.float32),
        compiler_params=pltpu.CompilerParams(
            dimension_semantics=("parallel",)),
    )(m, W1, b1r, W2b, b2p)


def kernel(x, emb, W1, b1, W2, b2):
    seq, batch = x.shape
    vocab, embed = emb.shape
    out_cols = W2.shape[1]

    npad_rows = 64
    pad_idx = vocab + (
        jnp.arange(batch * (SEQ_PAD - seq), dtype=jnp.int32)
        .reshape(batch, SEQ_PAD - seq) % npad_rows)
    xT = jnp.concatenate([x.astype(jnp.int32).T, pad_idx], axis=1)
    idx_flat = xT.reshape(-1)
    emb_pad = jnp.pad(emb.astype(jnp.bfloat16), ((0, npad_rows), (0, 0)))
    emb_packed = jax.lax.bitcast_convert_type(
        emb_pad.reshape(vocab + npad_rows, EWORDS, 2), jnp.int32)

    lane = jnp.arange(LANES)
    perm = jnp.concatenate(
        [jnp.concatenate([32 * k + 2 * lane, 32 * k + 2 * lane + 1])
         for k in range(WVECS)])
    W1p = W1[perm, :]

    opad = ((out_cols + 1279) // 1280) * 1280
    W2b = jnp.pad(W2, ((0, 0), (0, opad - out_cols))).astype(jnp.bfloat16)
    b2p = jnp.pad(b2, (0, opad - out_cols),
                  constant_values=-1e30).reshape(1, -1)
    sums = _pool(emb_packed, idx_flat, batch)
    return _mlp(sums, W1p, b1.reshape(1, -1), W2b, b2p, out_cols)
